# baseline (speedup 1.0000x reference)
import functools

import numpy as np

import jax
import jax.numpy as jnp
from jax import lax
from jax.experimental import pallas as pl
from jax.experimental.pallas import tpu as pltpu
from jax.experimental.pallas import tpu_sc as plsc

_H = 128
_N = 10000
_E = 160000
_NS = 16
_ND = 8

_NTILES = 32
_EPT = _E // _NTILES
_CB = 24
_EPT_PAD = 5016
_NCHUNK = _EPT_PAD // _CB
_DUMMY = 10200
_NPAD = 10240
_RPS = _NPAD // 16


def _table_body(hs_ref, hd_ref, xss_ref, xsd_ref, WsT, WdT, WaT, WbT, W1T, W2T,
                eb1, db1, pb1, pb2, tsrc_ref, tdst_ref):
    hs = hs_ref[...]
    s_ = xss_ref[...] @ WsT[...]
    a_ = hs @ WaT[...] + db1[...]
    v_ = jnp.maximum(hs @ W1T[...] + pb1[...], 0.0) @ W2T[...] + pb2[...]
    tsrc_ref[...] = jnp.concatenate([s_, a_, v_], axis=1)
    sd = xsd_ref[...] @ WdT[...] + eb1[...]
    b_ = hd_ref[...] @ WbT[...]
    tdst_ref[...] = jnp.concatenate([sd, b_], axis=1)


def _make_tables(hs, hd, xss, xsd, q):
    R = 2000
    full = lambda shp: pl.BlockSpec(shp, lambda i: (0, 0))
    row = lambda w: pl.BlockSpec((R, w), lambda i: (i, 0))
    return pl.pallas_call(
        _table_body,
        grid=(_N // R,),
        in_specs=[row(_H), row(_H), row(_NS), row(_NS),
                  full((_NS, _H)), full((_NS, _H)), full((_H, _H)),
                  full((_H, _H)), full((_H, _H)), full((_H, _H)),
                  full((1, _H)), full((1, _H)), full((1, _H)), full((1, _H))],
        out_specs=[row(3 * _H), row(2 * _H)],
        out_shape=[jax.ShapeDtypeStruct((_N, 3 * _H), jnp.float32),
                   jax.ShapeDtypeStruct((_N, 2 * _H), jnp.float32)],
    )(hs, hd, xss, xsd,
      q['ese_W1'][:, 4:20].T, q['ese_W1'][:, 20:36].T,
      q['dg_W1'][:, :_H].T, q['dg_W1'][:, _H:].T,
      q['pl_W1'].T, q['pl_W2'].T,
      q['ese_b1'][None, :], q['dg_b1'][None, :],
      q['pl_b1'][None, :], q['pl_b2'][None, :])


def _eap_body(ea_ref, WeaT, out_ref):
    out_ref[...] = ea_ref[...] @ WeaT[...]


def _make_eap(ea_perm, q):
    R = _EPT_PAD
    return pl.pallas_call(
        _eap_body,
        grid=(_NTILES,),
        in_specs=[pl.BlockSpec((R, 4), lambda i: (i, 0)),
                  pl.BlockSpec((4, _H), lambda i: (0, 0))],
        out_specs=pl.BlockSpec((R, _H), lambda i: (i, 0)),
        out_shape=jax.ShapeDtypeStruct((_NTILES * R, _H), jnp.float32),
    )(ea_perm, q['ese_W1'][:, :4].T)


def _sc_edges(ts1, td1, ep1, ei1, ts2, td2, ep2, ei2, ts3, td3, ep3, ei3,
              wball, wgall, call, zrows, p1, p2,
              idxA, idxB, rsA, rsB, rdA, rdB, eapA, eapB, m_v,
              wb_v, wg_v, consts_v, acc,
              semA0, semA1, semA2, semB0, semB1, semB2, semI0, semI1):
    c = lax.axis_index("c")
    s = lax.axis_index("s")
    w = s * 2 + c
    base_tile = w * _EPT_PAD
    row0 = w * _NCHUNK
    my_rows = pl.ds(s * _RPS, _RPS)

    idx = (idxA, idxB)
    rs = (rsA, rsB)
    rd = (rdA, rdB)
    eapb = (eapA, eapB)
    sems = ((semA0, semA1, semA2), (semB0, semB1, semB2))
    semi = (semI0, semI1)

    perms = [jnp.bitwise_xor(lax.iota(jnp.int32, 16), sh)[:, None]
             for sh in (8, 4, 2, 1)]
    _gdn = lax.GatherDimensionNumbers(
        offset_dims=(), collapsed_slice_dims=(0,), start_index_map=(0,))

    def lanesum(v):
        for i in perms:
            v = v + lax.gather(v, i, _gdn, (1,),
                               mode=lax.GatherScatterMode.PROMISE_IN_BOUNDS)
        return v

    def do_relation(tsrc, tdst, eap, eip, r):
        pltpu.sync_copy(wball.at[pl.ds(r * _H, _H)], wb_v)
        pltpu.sync_copy(wgall.at[pl.ds(r * _H, _H)], wg_v)
        pltpu.sync_copy(call.at[pl.ds(r * 16, 16)], consts_v)
        cv = consts_v[pl.ds(0, 16)]
        cb = cv[0]
        cg = cv[1]

        def load_idx(k, b, sem):
            pltpu.async_copy(eip.at[pl.ds(row0 + k, 1)], idx[b], sem)

        def wait_idx(b, sem):
            pltpu.make_async_copy(eip.at[pl.ds(row0, 1)], idx[b], sem).wait()


        def stage(k, b):
            pltpu.async_copy(tsrc.at[idx[b].at[0, 0]], rs[b], sems[b][0])
            pltpu.async_copy(tdst.at[idx[b].at[0, 1]], rd[b], sems[b][1])
            pltpu.async_copy(eap.at[pl.ds(base_tile + k * _CB, _CB)],
                             eapb[b], sems[b][2])

        def wait_gathers(b):
            pltpu.make_async_copy(tsrc.at[idx[b].at[0, 0]], rs[b], sems[b][0]).wait()
            pltpu.make_async_copy(tdst.at[idx[b].at[0, 1]], rd[b], sems[b][1]).wait()
            pltpu.make_async_copy(eap.at[pl.ds(0, _CB)], eapb[b], sems[b][2]).wait()

        def compute_scatter(b):
            rsb, rdb, epb = rs[b], rd[b], eapb[b]

            def edge2(j, carry2):
                for t in range(2):
                    e = 2 * j + t
                    acc1 = jnp.zeros((16,), jnp.float32)
                    acc2 = jnp.zeros((16,), jnp.float32)
                    for ci in range(8):
                        sl = pl.ds(ci * 16, 16)
                        sl2 = pl.ds(_H + ci * 16, 16)
                        z1 = epb[e, sl] + rsb[e, sl] + rdb[e, sl]
                        acc1 = acc1 + jnp.maximum(z1, 0.0) * wb_v[sl]
                        z2 = rsb[e, sl2] + rdb[e, sl2]
                        acc2 = acc2 + jnp.maximum(z2, 0.0) * wg_v[sl]
                    x1v = lanesum(acc1) + jnp.full((16,), cb, jnp.float32)
                    x2v = lanesum(acc2) + jnp.full((16,), cg, jnp.float32)
                    u = jnp.exp(-jnp.abs(x1v))
                    zz = u / (2.0 + u)
                    zq = zz * zz
                    poly = 1.0 + zq * (0.33333334 + zq * (0.2 + zq * (0.14285715 + zq * 0.11111111)))
                    sp = jnp.maximum(x1v, 0.0) + 2.0 * zz * poly
                    sg = 1.0 / (1.0 + jnp.exp(-x2v))
                    wv = sp * sg
                    for ci in range(8):
                        m_v[e, pl.ds(ci * 16, 16)] = wv * rsb[e, pl.ds(2 * _H + ci * 16, 16)]
                return carry2

            lax.fori_loop(0, _CB // 2, edge2, 0)
            pltpu.sync_copy(m_v, acc.at[idx[b].at[0, 1]], add=True)

        pltpu.sync_copy(eip.at[pl.ds(row0, 1)], idx[0])
        load_idx(1, 1, semi[1])
        stage(0, 0)

        def pair(p, carry):
            for b in range(2):
                k = 2 * p + b
                wait_idx(1 - b, semi[1 - b])
                stage(k + 1, 1 - b)
                wait_gathers(b)
                compute_scatter(b)
                nk = jnp.minimum(k + 2, _NCHUNK - 1)
                load_idx(nk, b, semi[b])
            return carry

        lax.fori_loop(0, (_NCHUNK - 1) // 2, pair, 0)
        wait_idx(1, semi[1])
        wait_gathers(0)
        compute_scatter(0)

    pltpu.sync_copy(zrows, acc.at[my_rows])
    plsc.subcore_barrier()
    do_relation(ts1, td1, ep1, ei1, 0)
    plsc.subcore_barrier()
    pltpu.sync_copy(acc.at[my_rows], p1.at[c, my_rows])
    pltpu.sync_copy(zrows, acc.at[my_rows])
    plsc.subcore_barrier()
    do_relation(ts2, td2, ep2, ei2, 1)
    do_relation(ts3, td3, ep3, ei3, 2)
    plsc.subcore_barrier()
    pltpu.sync_copy(acc.at[my_rows], p2.at[c, my_rows])


def _sc_call(tables, eis, wball, wgall, call, zrows):
    mesh = plsc.VectorSubcoreMesh(core_axis_name="c", subcore_axis_name="s")
    k = functools.partial(
        pl.kernel,
        out_type=[jax.ShapeDtypeStruct((2, _NPAD, _H), jnp.float32),
                  jax.ShapeDtypeStruct((2, _NPAD, _H), jnp.float32)],
        mesh=mesh,
        scratch_types=[
            pltpu.VMEM((1, 2, _CB), jnp.int32),
            pltpu.VMEM((1, 2, _CB), jnp.int32),
            pltpu.VMEM((_CB, 3 * _H), jnp.float32),
            pltpu.VMEM((_CB, 3 * _H), jnp.float32),
            pltpu.VMEM((_CB, 2 * _H), jnp.float32),
            pltpu.VMEM((_CB, 2 * _H), jnp.float32),
            pltpu.VMEM((_CB, _H), jnp.float32),
            pltpu.VMEM((_CB, _H), jnp.float32),
            pltpu.VMEM((_CB, _H), jnp.float32),
            pltpu.VMEM((_H,), jnp.float32),
            pltpu.VMEM((_H,), jnp.float32),
            pltpu.VMEM((16,), jnp.float32),
            pltpu.VMEM_SHARED((_NPAD, _H), jnp.float32),
            pltpu.SemaphoreType.DMA,
            pltpu.SemaphoreType.DMA,
            pltpu.SemaphoreType.DMA,
            pltpu.SemaphoreType.DMA,
            pltpu.SemaphoreType.DMA,
            pltpu.SemaphoreType.DMA,
            pltpu.SemaphoreType.DMA,
            pltpu.SemaphoreType.DMA,
        ],
    )(_sc_edges)
    (ts1, td1, ep1), (ts2, td2, ep2), (ts3, td3, ep3) = tables
    return k(ts1, td1, ep1, eis[0], ts2, td2, ep2, eis[1],
             ts3, td3, ep3, eis[2], wball, wgall, call, zrows)


def _node_body(p_ref, h_ref, dyn_ref, dpWT, WihT, WhhT, bih, bhh,
               mn_g, mn_b, dn_g, dn_b, hn_g, hn_b, rfWT, rf_b, dp_b,
               out_ref):
    def ln(x, g, b):
        mu = jnp.mean(x, axis=-1, keepdims=True)
        var = jnp.mean((x - mu) ** 2, axis=-1, keepdims=True)
        return (x - mu) / jnp.sqrt(var + 1e-5) * g + b

    M = p_ref[0, 0] + p_ref[0, 1]
    h = h_ref[0]
    m = ln(M, mn_g[0], mn_b[0])
    d = ln(dyn_ref[0] @ dpWT[0] + dp_b[0], dn_g[0], dn_b[0])
    x = jnp.concatenate([m, d], axis=1)
    gi = x @ WihT[0] + bih[0]
    gh = h @ WhhT[0] + bhh[0]
    r = jax.nn.sigmoid(gi[:, :_H] + gh[:, :_H])
    z = jax.nn.sigmoid(gi[:, _H:2 * _H] + gh[:, _H:2 * _H])
    n = jnp.tanh(gi[:, 2 * _H:] + r * gh[:, 2 * _H:])
    hn = (1.0 - z) * n + z * h
    out_ref[0] = ln(hn, hn_g[0], hn_b[0]) + m @ rfWT[0] + rf_b[0]


def _node_update(P, h_stack, dyn_stack, np_):
    R = 2000
    st = lambda shp: pl.BlockSpec((1,) + shp, lambda t, i: (t,) + (0,) * len(shp))
    return pl.pallas_call(
        _node_body,
        grid=(2, _N // R),
        in_specs=[pl.BlockSpec((1, 2, R, _H), lambda t, i: (t, 0, i, 0)),
                  pl.BlockSpec((1, R, _H), lambda t, i: (t, i, 0)),
                  pl.BlockSpec((1, R, _ND), lambda t, i: (t, i, 0)),
                  st((_ND, _H)), st((2 * _H, 3 * _H)), st((_H, 3 * _H)),
                  st((1, 3 * _H)), st((1, 3 * _H)),
                  st((1, _H)), st((1, _H)), st((1, _H)), st((1, _H)),
                  st((1, _H)), st((1, _H)), st((_H, _H)), st((1, _H)),
                  st((1, _H))],
        out_specs=pl.BlockSpec((1, R, _H), lambda t, i: (t, i, 0)),
        out_shape=jax.ShapeDtypeStruct((2, _N, _H), jnp.float32),
    )(P, h_stack, dyn_stack,
      np_('dp_W', lambda w: w.T), np_('Wih', lambda w: w.T), np_('Whh', lambda w: w.T),
      np_('bih', lambda w: w[None, :]), np_('bhh', lambda w: w[None, :]),
      np_('mn_g', lambda w: w[None, :]), np_('mn_b', lambda w: w[None, :]),
      np_('dn_g', lambda w: w[None, :]), np_('dn_b', lambda w: w[None, :]),
      np_('hn_g', lambda w: w[None, :]), np_('hn_b', lambda w: w[None, :]),
      np_('rf_W', lambda w: w.T), np_('rf_b', lambda w: w[None, :]),
      np_('dp_b', lambda w: w[None, :]))


_ea_perm = (np.arange(_NTILES)[:, None] * _EPT
            + np.minimum(np.arange(_EPT_PAD)[None, :], _EPT - 1)).reshape(-1)


def kernel(h_oneD, h_twoD, dyn_oneD, dyn_twoD, xs_oneD, xs_twoD,
           ea_r1, ea_r2, ea_r3, ei_r1, ei_r2, ei_r3, params):
    pr = params['rel']
    pn = params['node']
    rels = [(pr['r1'], h_oneD, h_oneD, xs_oneD, xs_oneD, ea_r1, ei_r1),
            (pr['r2'], h_twoD, h_twoD, xs_twoD, xs_twoD, ea_r2, ei_r2),
            (pr['r3'], h_oneD, h_twoD, xs_oneD, xs_twoD, ea_r3, ei_r3)]

    tables, eis, wbs, wgs, cs = [], [], [], [], []
    for q, hs, hd, xss, xsd, ea, ei in rels:
        tsrc, tdst = _make_tables(hs, hd, xss, xsd, q)
        eap = _make_eap(ea[_ea_perm], q)
        tables.append((tsrc, tdst, eap))
        npad = _EPT_PAD - _EPT
        es = jnp.pad(ei[0].reshape(_NTILES, _EPT), ((0, 0), (0, npad)))
        ed = jnp.pad(ei[1].reshape(_NTILES, _EPT), ((0, 0), (0, npad)),
                     constant_values=_DUMMY)
        es = es.reshape(_NTILES, _NCHUNK, _CB)
        ed = ed.reshape(_NTILES, _NCHUNK, _CB)
        eis.append(jnp.stack([es, ed], axis=2).reshape(_NTILES * _NCHUNK, 2, _CB))
        wbs.append(q['ese_W2'].T @ q['bw_W'][0])
        wgs.append(q['dg_W2'][0])
        cb = q['ese_b2'] @ q['bw_W'][0] + q['bw_b'][0]
        cg = q['dg_b2'][0]
        cs.append(jnp.zeros((16,), jnp.float32).at[0].set(cb).at[1].set(cg))

    wball = jnp.concatenate(wbs)
    wgall = jnp.concatenate(wgs)
    call = jnp.concatenate(cs)
    zrows = jnp.zeros((_RPS, _H), jnp.float32)

    p1, p2 = _sc_call(tables, eis, wball, wgall, call, zrows)

    P = jnp.stack([p1, p2])
    h_stack = jnp.stack([h_oneD, h_twoD])
    dyn_stack = jnp.stack([dyn_oneD, dyn_twoD])

    def np_(name, f):
        return jnp.stack([f(pn['oneD'][name]), f(pn['twoD'][name])])

    out = _node_update(P, h_stack, dyn_stack, np_)
    return out.reshape(2 * _N, _H)


# speedup vs baseline: 1.0111x; 1.0111x over previous
import functools

import numpy as np

import jax
import jax.numpy as jnp
from jax import lax
from jax.experimental import pallas as pl
from jax.experimental.pallas import tpu as pltpu
from jax.experimental.pallas import tpu_sc as plsc

_H = 128
_N = 10000
_E = 160000
_NS = 16
_ND = 8

_NTILES = 32
_EPT = _E // _NTILES
_CB = 24
_EPT_PAD = 5016
_NCHUNK = _EPT_PAD // _CB
_DUMMY = 10200
_NPAD = 10240
_RPS = _NPAD // 16


def _table_body(hs_ref, hd_ref, xss_ref, xsd_ref, WsT, WdT, WaT, WbT, W1T, W2T,
                eb1, db1, pb1, pb2, tsrc_ref, tdst_ref):
    hs = hs_ref[...]
    s_ = xss_ref[...] @ WsT[...]
    a_ = hs @ WaT[...] + db1[...]
    v_ = jnp.maximum(hs @ W1T[...] + pb1[...], 0.0) @ W2T[...] + pb2[...]
    tsrc_ref[...] = jnp.concatenate([s_, a_, v_], axis=1)
    sd = xsd_ref[...] @ WdT[...] + eb1[...]
    b_ = hd_ref[...] @ WbT[...]
    tdst_ref[...] = jnp.concatenate([sd, b_], axis=1)


def _make_tables(hs, hd, xss, xsd, q):
    R = 2000
    full = lambda shp: pl.BlockSpec(shp, lambda i: (0, 0))
    row = lambda w: pl.BlockSpec((R, w), lambda i: (i, 0))
    return pl.pallas_call(
        _table_body,
        grid=(_N // R,),
        in_specs=[row(_H), row(_H), row(_NS), row(_NS),
                  full((_NS, _H)), full((_NS, _H)), full((_H, _H)),
                  full((_H, _H)), full((_H, _H)), full((_H, _H)),
                  full((1, _H)), full((1, _H)), full((1, _H)), full((1, _H))],
        out_specs=[row(3 * _H), row(2 * _H)],
        out_shape=[jax.ShapeDtypeStruct((_N, 3 * _H), jnp.float32),
                   jax.ShapeDtypeStruct((_N, 2 * _H), jnp.float32)],
    )(hs, hd, xss, xsd,
      q['ese_W1'][:, 4:20].T, q['ese_W1'][:, 20:36].T,
      q['dg_W1'][:, :_H].T, q['dg_W1'][:, _H:].T,
      q['pl_W1'].T, q['pl_W2'].T,
      q['ese_b1'][None, :], q['dg_b1'][None, :],
      q['pl_b1'][None, :], q['pl_b2'][None, :])


def _eap_body(ea_ref, WeaT, out_ref):
    out_ref[...] = ea_ref[...] @ WeaT[...]


def _make_eap(ea_perm, q):
    R = _EPT_PAD
    return pl.pallas_call(
        _eap_body,
        grid=(_NTILES,),
        in_specs=[pl.BlockSpec((R, 4), lambda i: (i, 0)),
                  pl.BlockSpec((4, _H), lambda i: (0, 0))],
        out_specs=pl.BlockSpec((R, _H), lambda i: (i, 0)),
        out_shape=jax.ShapeDtypeStruct((_NTILES * R, _H), jnp.float32),
    )(ea_perm, q['ese_W1'][:, :4].T)


def _sc_edges(ts1, td1, ep1, ei1, ts2, td2, ep2, ei2, ts3, td3, ep3, ei3,
              wball, wgall, call, zrows, p1, p2,
              idxA, idxB, rsA, rsB, rdA, rdB, eapA, eapB, m_v,
              wb_v, wg_v, consts_v, acc,
              semA0, semA1, semA2, semB0, semB1, semB2, semI0, semI1):
    c = lax.axis_index("c")
    s = lax.axis_index("s")
    w = s * 2 + c
    base_tile = w * _EPT_PAD
    row0 = w * _NCHUNK
    my_rows = pl.ds(s * _RPS, _RPS)

    idx = (idxA, idxB)
    rs = (rsA, rsB)
    rd = (rdA, rdB)
    eapb = (eapA, eapB)
    sems = ((semA0, semA1, semA2), (semB0, semB1, semB2))
    semi = (semI0, semI1)

    perms = [jnp.bitwise_xor(lax.iota(jnp.int32, 16), sh)[:, None]
             for sh in (8, 4, 2, 1)]
    _gdn = lax.GatherDimensionNumbers(
        offset_dims=(), collapsed_slice_dims=(0,), start_index_map=(0,))

    def lanesum(v):
        for i in perms:
            v = v + lax.gather(v, i, _gdn, (1,),
                               mode=lax.GatherScatterMode.PROMISE_IN_BOUNDS)
        return v

    def do_relation(tsrc, tdst, eap, eip, r):
        pltpu.sync_copy(wball.at[pl.ds(r * _H, _H)], wb_v)
        pltpu.sync_copy(wgall.at[pl.ds(r * _H, _H)], wg_v)
        pltpu.sync_copy(call.at[pl.ds(r * 16, 16)], consts_v)
        cv = consts_v[pl.ds(0, 16)]
        cb = cv[0]
        cg = cv[1]

        def load_idx(k, b, sem):
            pltpu.async_copy(eip.at[pl.ds(row0 + k, 1)], idx[b], sem)

        def wait_idx(b, sem):
            pltpu.make_async_copy(eip.at[pl.ds(row0, 1)], idx[b], sem).wait()


        def stage(k, b):
            pltpu.async_copy(tsrc.at[idx[b].at[0, 0]], rs[b], sems[b][0])
            pltpu.async_copy(tdst.at[idx[b].at[0, 1]], rd[b], sems[b][1])
            pltpu.async_copy(eap.at[pl.ds(base_tile + k * _CB, _CB)],
                             eapb[b], sems[b][2])

        def wait_gathers(b):
            pltpu.make_async_copy(tsrc.at[idx[b].at[0, 0]], rs[b], sems[b][0]).wait()
            pltpu.make_async_copy(tdst.at[idx[b].at[0, 1]], rd[b], sems[b][1]).wait()
            pltpu.make_async_copy(eap.at[pl.ds(0, _CB)], eapb[b], sems[b][2]).wait()

        def compute_scatter(b):
            rsb, rdb, epb = rs[b], rd[b], eapb[b]
            wbs = [wb_v[pl.ds(ci * 16, 16)] for ci in range(8)]
            wgs = [wg_v[pl.ds(ci * 16, 16)] for ci in range(8)]
            cbv = jnp.full((16,), cb, jnp.float32)
            cgv = jnp.full((16,), cg, jnp.float32)

            def edge4(j, carry2):
                for t in range(4):
                    e = 4 * j + t
                    acc1 = jnp.zeros((16,), jnp.float32)
                    acc2 = jnp.zeros((16,), jnp.float32)
                    for ci in range(8):
                        sl = pl.ds(ci * 16, 16)
                        sl2 = pl.ds(_H + ci * 16, 16)
                        z1 = epb[e, sl] + rsb[e, sl] + rdb[e, sl]
                        acc1 = acc1 + jnp.maximum(z1, 0.0) * wbs[ci]
                        z2 = rsb[e, sl2] + rdb[e, sl2]
                        acc2 = acc2 + jnp.maximum(z2, 0.0) * wgs[ci]
                    x1v = lanesum(acc1) + cbv
                    x2v = lanesum(acc2) + cgv
                    u = jnp.exp(-jnp.abs(x1v))
                    zz = u / (2.0 + u)
                    zq = zz * zz
                    poly = 1.0 + zq * (0.33333334 + zq * (0.2 + zq * (0.14285715 + zq * 0.11111111)))
                    sp = jnp.maximum(x1v, 0.0) + 2.0 * zz * poly
                    sg = 1.0 / (1.0 + jnp.exp(-x2v))
                    wv = sp * sg
                    for ci in range(8):
                        m_v[e, pl.ds(ci * 16, 16)] = wv * rsb[e, pl.ds(2 * _H + ci * 16, 16)]
                return carry2

            lax.fori_loop(0, _CB // 4, edge4, 0)
            pltpu.sync_copy(m_v, acc.at[idx[b].at[0, 1]], add=True)

        pltpu.sync_copy(eip.at[pl.ds(row0, 1)], idx[0])
        load_idx(1, 1, semi[1])
        stage(0, 0)

        def pair(p, carry):
            for b in range(2):
                k = 2 * p + b
                wait_idx(1 - b, semi[1 - b])
                stage(k + 1, 1 - b)
                wait_gathers(b)
                compute_scatter(b)
                nk = jnp.minimum(k + 2, _NCHUNK - 1)
                load_idx(nk, b, semi[b])
            return carry

        lax.fori_loop(0, (_NCHUNK - 1) // 2, pair, 0)
        wait_idx(1, semi[1])
        wait_gathers(0)
        compute_scatter(0)

    pltpu.sync_copy(zrows, acc.at[my_rows])
    plsc.subcore_barrier()
    do_relation(ts1, td1, ep1, ei1, 0)
    plsc.subcore_barrier()
    pltpu.sync_copy(acc.at[my_rows], p1.at[c, my_rows])
    pltpu.sync_copy(zrows, acc.at[my_rows])
    plsc.subcore_barrier()
    do_relation(ts2, td2, ep2, ei2, 1)
    do_relation(ts3, td3, ep3, ei3, 2)
    plsc.subcore_barrier()
    pltpu.sync_copy(acc.at[my_rows], p2.at[c, my_rows])


def _sc_call(tables, eis, wball, wgall, call, zrows):
    mesh = plsc.VectorSubcoreMesh(core_axis_name="c", subcore_axis_name="s")
    k = functools.partial(
        pl.kernel,
        out_type=[jax.ShapeDtypeStruct((2, _NPAD, _H), jnp.float32),
                  jax.ShapeDtypeStruct((2, _NPAD, _H), jnp.float32)],
        mesh=mesh,
        scratch_types=[
            pltpu.VMEM((1, 2, _CB), jnp.int32),
            pltpu.VMEM((1, 2, _CB), jnp.int32),
            pltpu.VMEM((_CB, 3 * _H), jnp.float32),
            pltpu.VMEM((_CB, 3 * _H), jnp.float32),
            pltpu.VMEM((_CB, 2 * _H), jnp.float32),
            pltpu.VMEM((_CB, 2 * _H), jnp.float32),
            pltpu.VMEM((_CB, _H), jnp.float32),
            pltpu.VMEM((_CB, _H), jnp.float32),
            pltpu.VMEM((_CB, _H), jnp.float32),
            pltpu.VMEM((_H,), jnp.float32),
            pltpu.VMEM((_H,), jnp.float32),
            pltpu.VMEM((16,), jnp.float32),
            pltpu.VMEM_SHARED((_NPAD, _H), jnp.float32),
            pltpu.SemaphoreType.DMA,
            pltpu.SemaphoreType.DMA,
            pltpu.SemaphoreType.DMA,
            pltpu.SemaphoreType.DMA,
            pltpu.SemaphoreType.DMA,
            pltpu.SemaphoreType.DMA,
            pltpu.SemaphoreType.DMA,
            pltpu.SemaphoreType.DMA,
        ],
    )(_sc_edges)
    (ts1, td1, ep1), (ts2, td2, ep2), (ts3, td3, ep3) = tables
    return k(ts1, td1, ep1, eis[0], ts2, td2, ep2, eis[1],
             ts3, td3, ep3, eis[2], wball, wgall, call, zrows)


def _node_body(p_ref, h_ref, dyn_ref, dpWT, WihT, WhhT, bih, bhh,
               mn_g, mn_b, dn_g, dn_b, hn_g, hn_b, rfWT, rf_b, dp_b,
               out_ref):
    def ln(x, g, b):
        mu = jnp.mean(x, axis=-1, keepdims=True)
        var = jnp.mean((x - mu) ** 2, axis=-1, keepdims=True)
        return (x - mu) / jnp.sqrt(var + 1e-5) * g + b

    M = p_ref[0, 0] + p_ref[0, 1]
    h = h_ref[0]
    m = ln(M, mn_g[0], mn_b[0])
    d = ln(dyn_ref[0] @ dpWT[0] + dp_b[0], dn_g[0], dn_b[0])
    x = jnp.concatenate([m, d], axis=1)
    gi = x @ WihT[0] + bih[0]
    gh = h @ WhhT[0] + bhh[0]
    r = jax.nn.sigmoid(gi[:, :_H] + gh[:, :_H])
    z = jax.nn.sigmoid(gi[:, _H:2 * _H] + gh[:, _H:2 * _H])
    n = jnp.tanh(gi[:, 2 * _H:] + r * gh[:, 2 * _H:])
    hn = (1.0 - z) * n + z * h
    out_ref[0] = ln(hn, hn_g[0], hn_b[0]) + m @ rfWT[0] + rf_b[0]


def _node_update(P, h_stack, dyn_stack, np_):
    R = 2000
    st = lambda shp: pl.BlockSpec((1,) + shp, lambda t, i: (t,) + (0,) * len(shp))
    return pl.pallas_call(
        _node_body,
        grid=(2, _N // R),
        in_specs=[pl.BlockSpec((1, 2, R, _H), lambda t, i: (t, 0, i, 0)),
                  pl.BlockSpec((1, R, _H), lambda t, i: (t, i, 0)),
                  pl.BlockSpec((1, R, _ND), lambda t, i: (t, i, 0)),
                  st((_ND, _H)), st((2 * _H, 3 * _H)), st((_H, 3 * _H)),
                  st((1, 3 * _H)), st((1, 3 * _H)),
                  st((1, _H)), st((1, _H)), st((1, _H)), st((1, _H)),
                  st((1, _H)), st((1, _H)), st((_H, _H)), st((1, _H)),
                  st((1, _H))],
        out_specs=pl.BlockSpec((1, R, _H), lambda t, i: (t, i, 0)),
        out_shape=jax.ShapeDtypeStruct((2, _N, _H), jnp.float32),
    )(P, h_stack, dyn_stack,
      np_('dp_W', lambda w: w.T), np_('Wih', lambda w: w.T), np_('Whh', lambda w: w.T),
      np_('bih', lambda w: w[None, :]), np_('bhh', lambda w: w[None, :]),
      np_('mn_g', lambda w: w[None, :]), np_('mn_b', lambda w: w[None, :]),
      np_('dn_g', lambda w: w[None, :]), np_('dn_b', lambda w: w[None, :]),
      np_('hn_g', lambda w: w[None, :]), np_('hn_b', lambda w: w[None, :]),
      np_('rf_W', lambda w: w.T), np_('rf_b', lambda w: w[None, :]),
      np_('dp_b', lambda w: w[None, :]))


_ea_perm = (np.arange(_NTILES)[:, None] * _EPT
            + np.minimum(np.arange(_EPT_PAD)[None, :], _EPT - 1)).reshape(-1)


def kernel(h_oneD, h_twoD, dyn_oneD, dyn_twoD, xs_oneD, xs_twoD,
           ea_r1, ea_r2, ea_r3, ei_r1, ei_r2, ei_r3, params):
    pr = params['rel']
    pn = params['node']
    rels = [(pr['r1'], h_oneD, h_oneD, xs_oneD, xs_oneD, ea_r1, ei_r1),
            (pr['r2'], h_twoD, h_twoD, xs_twoD, xs_twoD, ea_r2, ei_r2),
            (pr['r3'], h_oneD, h_twoD, xs_oneD, xs_twoD, ea_r3, ei_r3)]

    tables, eis, wbs, wgs, cs = [], [], [], [], []
    for q, hs, hd, xss, xsd, ea, ei in rels:
        tsrc, tdst = _make_tables(hs, hd, xss, xsd, q)
        eap = _make_eap(ea[_ea_perm], q)
        tables.append((tsrc, tdst, eap))
        npad = _EPT_PAD - _EPT
        es = jnp.pad(ei[0].reshape(_NTILES, _EPT), ((0, 0), (0, npad)))
        ed = jnp.pad(ei[1].reshape(_NTILES, _EPT), ((0, 0), (0, npad)),
                     constant_values=_DUMMY)
        es = es.reshape(_NTILES, _NCHUNK, _CB)
        ed = ed.reshape(_NTILES, _NCHUNK, _CB)
        eis.append(jnp.stack([es, ed], axis=2).reshape(_NTILES * _NCHUNK, 2, _CB))
        wbs.append(q['ese_W2'].T @ q['bw_W'][0])
        wgs.append(q['dg_W2'][0])
        cb = q['ese_b2'] @ q['bw_W'][0] + q['bw_b'][0]
        cg = q['dg_b2'][0]
        cs.append(jnp.zeros((16,), jnp.float32).at[0].set(cb).at[1].set(cg))

    wball = jnp.concatenate(wbs)
    wgall = jnp.concatenate(wgs)
    call = jnp.concatenate(cs)
    zrows = jnp.zeros((_RPS, _H), jnp.float32)

    p1, p2 = _sc_call(tables, eis, wball, wgall, call, zrows)

    P = jnp.stack([p1, p2])
    h_stack = jnp.stack([h_oneD, h_twoD])
    dyn_stack = jnp.stack([dyn_oneD, dyn_twoD])

    def np_(name, f):
        return jnp.stack([f(pn['oneD'][name]), f(pn['twoD'][name])])

    out = _node_update(P, h_stack, dyn_stack, np_)
    return out.reshape(2 * _N, _H)


# speedup vs baseline: 1.0435x; 1.0320x over previous
import functools

import numpy as np

import jax
import jax.numpy as jnp
from jax import lax
from jax.experimental import pallas as pl
from jax.experimental.pallas import tpu as pltpu
from jax.experimental.pallas import tpu_sc as plsc

_H = 128
_N = 10000
_E = 160000
_NS = 16
_ND = 8

_NTILES = 32
_EPT = _E // _NTILES
_CB = 16
_EPT_PAD = 5008
_NCHUNK = _EPT_PAD // _CB
_DUMMY = 10200
_NPAD = 10240
_RPS = _NPAD // 16


def _table_body(hs_ref, hd_ref, xss_ref, xsd_ref, WsT, WdT, WaT, WbT, W1T, W2T,
                eb1, db1, pb1, pb2, tsrc_ref, tdst_ref):
    hs = hs_ref[...]
    s_ = xss_ref[...] @ WsT[...]
    a_ = hs @ WaT[...] + db1[...]
    v_ = jnp.maximum(hs @ W1T[...] + pb1[...], 0.0) @ W2T[...] + pb2[...]
    tsrc_ref[...] = jnp.concatenate([s_, a_, v_], axis=1)
    sd = xsd_ref[...] @ WdT[...] + eb1[...]
    b_ = hd_ref[...] @ WbT[...]
    tdst_ref[...] = jnp.concatenate([sd, b_], axis=1)


def _make_tables(hs, hd, xss, xsd, q):
    R = 2000
    full = lambda shp: pl.BlockSpec(shp, lambda i: (0, 0))
    row = lambda w: pl.BlockSpec((R, w), lambda i: (i, 0))
    return pl.pallas_call(
        _table_body,
        grid=(_N // R,),
        in_specs=[row(_H), row(_H), row(_NS), row(_NS),
                  full((_NS, _H)), full((_NS, _H)), full((_H, _H)),
                  full((_H, _H)), full((_H, _H)), full((_H, _H)),
                  full((1, _H)), full((1, _H)), full((1, _H)), full((1, _H))],
        out_specs=[row(3 * _H), row(2 * _H)],
        out_shape=[jax.ShapeDtypeStruct((_N, 3 * _H), jnp.float32),
                   jax.ShapeDtypeStruct((_N, 2 * _H), jnp.float32)],
    )(hs, hd, xss, xsd,
      q['ese_W1'][:, 4:20].T, q['ese_W1'][:, 20:36].T,
      q['dg_W1'][:, :_H].T, q['dg_W1'][:, _H:].T,
      q['pl_W1'].T, q['pl_W2'].T,
      q['ese_b1'][None, :], q['dg_b1'][None, :],
      q['pl_b1'][None, :], q['pl_b2'][None, :])


def _eap_body(ea_ref, WeaT, out_ref):
    out_ref[...] = ea_ref[...] @ WeaT[...]


def _make_eap(ea_perm, q):
    R = _EPT_PAD
    return pl.pallas_call(
        _eap_body,
        grid=(_NTILES,),
        in_specs=[pl.BlockSpec((R, 4), lambda i: (i, 0)),
                  pl.BlockSpec((4, _H), lambda i: (0, 0))],
        out_specs=pl.BlockSpec((R, _H), lambda i: (i, 0)),
        out_shape=jax.ShapeDtypeStruct((_NTILES * R, _H), jnp.float32),
    )(ea_perm, q['ese_W1'][:, :4].T)


def _sc_edges(ts1, td1, ep1, ei1, ts2, td2, ep2, ei2, ts3, td3, ep3, ei3,
              wball, wgall, call, zrows, p1, p2,
              idxA, idxB, rsA, rsB, rdA, rdB, eapA, eapB, m_v,
              wb_v, wg_v, consts_v, acc,
              semA0, semA1, semA2, semB0, semB1, semB2, semI0, semI1):
    c = lax.axis_index("c")
    s = lax.axis_index("s")
    w = s * 2 + c
    base_tile = w * _EPT_PAD
    row0 = w * _NCHUNK
    my_rows = pl.ds(s * _RPS, _RPS)

    idx = (idxA, idxB)
    rs = (rsA, rsB)
    rd = (rdA, rdB)
    eapb = (eapA, eapB)
    sems = ((semA0, semA1, semA2), (semB0, semB1, semB2))
    semi = (semI0, semI1)

    perms = [jnp.bitwise_xor(lax.iota(jnp.int32, 16), sh)[:, None]
             for sh in (8, 4, 2, 1)]
    _gdn = lax.GatherDimensionNumbers(
        offset_dims=(), collapsed_slice_dims=(0,), start_index_map=(0,))

    def lanesum(v):
        for i in perms:
            v = v + lax.gather(v, i, _gdn, (1,),
                               mode=lax.GatherScatterMode.PROMISE_IN_BOUNDS)
        return v

    def do_relation(tsrc, tdst, eap, eip, r):
        pltpu.sync_copy(wball.at[pl.ds(r * _H, _H)], wb_v)
        pltpu.sync_copy(wgall.at[pl.ds(r * _H, _H)], wg_v)
        pltpu.sync_copy(call.at[pl.ds(r * 16, 16)], consts_v)
        cv = consts_v[pl.ds(0, 16)]
        cb = cv[0]
        cg = cv[1]

        def load_idx(k, b, sem):
            pltpu.async_copy(eip.at[pl.ds(row0 + k, 1)], idx[b], sem)

        def wait_idx(b, sem):
            pltpu.make_async_copy(eip.at[pl.ds(row0, 1)], idx[b], sem).wait()


        def stage(k, b):
            pltpu.async_copy(tsrc.at[idx[b].at[0, 0]], rs[b], sems[b][0])
            pltpu.async_copy(tdst.at[idx[b].at[0, 1]], rd[b], sems[b][1])
            pltpu.async_copy(eap.at[pl.ds(base_tile + k * _CB, _CB)],
                             eapb[b], sems[b][2])

        def wait_gathers(b):
            pltpu.make_async_copy(tsrc.at[idx[b].at[0, 0]], rs[b], sems[b][0]).wait()
            pltpu.make_async_copy(tdst.at[idx[b].at[0, 1]], rd[b], sems[b][1]).wait()
            pltpu.make_async_copy(eap.at[pl.ds(0, _CB)], eapb[b], sems[b][2]).wait()

        def compute_scatter(b):
            rsb, rdb, epb = rs[b], rd[b], eapb[b]
            cbv = jnp.full((16,), cb, jnp.float32)
            cgv = jnp.full((16,), cg, jnp.float32)
            lane = lax.iota(jnp.int32, 16)

            def edge_dot(j, carry):
                s1_all, s2_all = carry
                for t in range(2):
                    e = 2 * j + t
                    acc1 = jnp.zeros((16,), jnp.float32)
                    acc2 = jnp.zeros((16,), jnp.float32)
                    for ci in range(8):
                        sl = pl.ds(ci * 16, 16)
                        sl2 = pl.ds(_H + ci * 16, 16)
                        z1 = epb[e, sl] + rsb[e, sl] + rdb[e, sl]
                        acc1 = acc1 + jnp.maximum(z1, 0.0) * wb_v[sl]
                        z2 = rsb[e, sl2] + rdb[e, sl2]
                        acc2 = acc2 + jnp.maximum(z2, 0.0) * wg_v[sl]
                    sel = lane == e
                    s1_all = jnp.where(sel, lanesum(acc1), s1_all)
                    s2_all = jnp.where(sel, lanesum(acc2), s2_all)
                return (s1_all, s2_all)

            s1_all, s2_all = lax.fori_loop(
                0, _CB // 2, edge_dot,
                (jnp.zeros((16,), jnp.float32), jnp.zeros((16,), jnp.float32)))

            y1 = s1_all + cbv
            y2 = s2_all + cgv
            u = jnp.exp(-jnp.abs(y1))
            zz = u / (2.0 + u)
            zq = zz * zz
            poly = 1.0 + zq * (0.33333334 + zq * (0.2 + zq * (0.14285715 + zq * 0.11111111)))
            sp = jnp.maximum(y1, 0.0) + 2.0 * zz * poly
            sg = 1.0 / (1.0 + jnp.exp(-y2))
            wv_all = sp * sg

            def edge_m(j, carry):
                for t in range(2):
                    e = 2 * j + t
                    wv = lanesum(jnp.where(lane == e, wv_all, 0.0))
                    for ci in range(8):
                        m_v[e, pl.ds(ci * 16, 16)] = wv * rsb[e, pl.ds(2 * _H + ci * 16, 16)]
                return carry

            lax.fori_loop(0, _CB // 2, edge_m, 0)

            pltpu.sync_copy(m_v, acc.at[idx[b].at[0, 1]], add=True)

        pltpu.sync_copy(eip.at[pl.ds(row0, 1)], idx[0])
        load_idx(1, 1, semi[1])
        stage(0, 0)

        def pair(p, carry):
            for b in range(2):
                k = 2 * p + b
                wait_idx(1 - b, semi[1 - b])
                stage(k + 1, 1 - b)
                wait_gathers(b)
                compute_scatter(b)
                nk = jnp.minimum(k + 2, _NCHUNK - 1)
                load_idx(nk, b, semi[b])
            return carry

        lax.fori_loop(0, (_NCHUNK - 1) // 2, pair, 0)
        wait_idx(1, semi[1])
        wait_gathers(0)
        compute_scatter(0)

    pltpu.sync_copy(zrows, acc.at[my_rows])
    plsc.subcore_barrier()
    do_relation(ts1, td1, ep1, ei1, 0)
    plsc.subcore_barrier()
    pltpu.sync_copy(acc.at[my_rows], p1.at[c, my_rows])
    pltpu.sync_copy(zrows, acc.at[my_rows])
    plsc.subcore_barrier()
    do_relation(ts2, td2, ep2, ei2, 1)
    do_relation(ts3, td3, ep3, ei3, 2)
    plsc.subcore_barrier()
    pltpu.sync_copy(acc.at[my_rows], p2.at[c, my_rows])


def _sc_call(tables, eis, wball, wgall, call, zrows):
    mesh = plsc.VectorSubcoreMesh(core_axis_name="c", subcore_axis_name="s")
    k = functools.partial(
        pl.kernel,
        out_type=[jax.ShapeDtypeStruct((2, _NPAD, _H), jnp.float32),
                  jax.ShapeDtypeStruct((2, _NPAD, _H), jnp.float32)],
        mesh=mesh,
        scratch_types=[
            pltpu.VMEM((1, 2, _CB), jnp.int32),
            pltpu.VMEM((1, 2, _CB), jnp.int32),
            pltpu.VMEM((_CB, 3 * _H), jnp.float32),
            pltpu.VMEM((_CB, 3 * _H), jnp.float32),
            pltpu.VMEM((_CB, 2 * _H), jnp.float32),
            pltpu.VMEM((_CB, 2 * _H), jnp.float32),
            pltpu.VMEM((_CB, _H), jnp.float32),
            pltpu.VMEM((_CB, _H), jnp.float32),
            pltpu.VMEM((_CB, _H), jnp.float32),
            pltpu.VMEM((_H,), jnp.float32),
            pltpu.VMEM((_H,), jnp.float32),
            pltpu.VMEM((16,), jnp.float32),
            pltpu.VMEM_SHARED((_NPAD, _H), jnp.float32),
            pltpu.SemaphoreType.DMA,
            pltpu.SemaphoreType.DMA,
            pltpu.SemaphoreType.DMA,
            pltpu.SemaphoreType.DMA,
            pltpu.SemaphoreType.DMA,
            pltpu.SemaphoreType.DMA,
            pltpu.SemaphoreType.DMA,
            pltpu.SemaphoreType.DMA,
        ],
    )(_sc_edges)
    (ts1, td1, ep1), (ts2, td2, ep2), (ts3, td3, ep3) = tables
    return k(ts1, td1, ep1, eis[0], ts2, td2, ep2, eis[1],
             ts3, td3, ep3, eis[2], wball, wgall, call, zrows)


def _node_body(p_ref, h_ref, dyn_ref, dpWT, WihT, WhhT, bih, bhh,
               mn_g, mn_b, dn_g, dn_b, hn_g, hn_b, rfWT, rf_b, dp_b,
               out_ref):
    def ln(x, g, b):
        mu = jnp.mean(x, axis=-1, keepdims=True)
        var = jnp.mean((x - mu) ** 2, axis=-1, keepdims=True)
        return (x - mu) / jnp.sqrt(var + 1e-5) * g + b

    M = p_ref[0, 0] + p_ref[0, 1]
    h = h_ref[0]
    m = ln(M, mn_g[0], mn_b[0])
    d = ln(dyn_ref[0] @ dpWT[0] + dp_b[0], dn_g[0], dn_b[0])
    x = jnp.concatenate([m, d], axis=1)
    gi = x @ WihT[0] + bih[0]
    gh = h @ WhhT[0] + bhh[0]
    r = jax.nn.sigmoid(gi[:, :_H] + gh[:, :_H])
    z = jax.nn.sigmoid(gi[:, _H:2 * _H] + gh[:, _H:2 * _H])
    n = jnp.tanh(gi[:, 2 * _H:] + r * gh[:, 2 * _H:])
    hn = (1.0 - z) * n + z * h
    out_ref[0] = ln(hn, hn_g[0], hn_b[0]) + m @ rfWT[0] + rf_b[0]


def _node_update(P, h_stack, dyn_stack, np_):
    R = 2000
    st = lambda shp: pl.BlockSpec((1,) + shp, lambda t, i: (t,) + (0,) * len(shp))
    return pl.pallas_call(
        _node_body,
        grid=(2, _N // R),
        in_specs=[pl.BlockSpec((1, 2, R, _H), lambda t, i: (t, 0, i, 0)),
                  pl.BlockSpec((1, R, _H), lambda t, i: (t, i, 0)),
                  pl.BlockSpec((1, R, _ND), lambda t, i: (t, i, 0)),
                  st((_ND, _H)), st((2 * _H, 3 * _H)), st((_H, 3 * _H)),
                  st((1, 3 * _H)), st((1, 3 * _H)),
                  st((1, _H)), st((1, _H)), st((1, _H)), st((1, _H)),
                  st((1, _H)), st((1, _H)), st((_H, _H)), st((1, _H)),
                  st((1, _H))],
        out_specs=pl.BlockSpec((1, R, _H), lambda t, i: (t, i, 0)),
        out_shape=jax.ShapeDtypeStruct((2, _N, _H), jnp.float32),
    )(P, h_stack, dyn_stack,
      np_('dp_W', lambda w: w.T), np_('Wih', lambda w: w.T), np_('Whh', lambda w: w.T),
      np_('bih', lambda w: w[None, :]), np_('bhh', lambda w: w[None, :]),
      np_('mn_g', lambda w: w[None, :]), np_('mn_b', lambda w: w[None, :]),
      np_('dn_g', lambda w: w[None, :]), np_('dn_b', lambda w: w[None, :]),
      np_('hn_g', lambda w: w[None, :]), np_('hn_b', lambda w: w[None, :]),
      np_('rf_W', lambda w: w.T), np_('rf_b', lambda w: w[None, :]),
      np_('dp_b', lambda w: w[None, :]))


_ea_perm = (np.arange(_NTILES)[:, None] * _EPT
            + np.minimum(np.arange(_EPT_PAD)[None, :], _EPT - 1)).reshape(-1)


def kernel(h_oneD, h_twoD, dyn_oneD, dyn_twoD, xs_oneD, xs_twoD,
           ea_r1, ea_r2, ea_r3, ei_r1, ei_r2, ei_r3, params):
    pr = params['rel']
    pn = params['node']
    rels = [(pr['r1'], h_oneD, h_oneD, xs_oneD, xs_oneD, ea_r1, ei_r1),
            (pr['r2'], h_twoD, h_twoD, xs_twoD, xs_twoD, ea_r2, ei_r2),
            (pr['r3'], h_oneD, h_twoD, xs_oneD, xs_twoD, ea_r3, ei_r3)]

    tables, eis, wbs, wgs, cs = [], [], [], [], []
    for q, hs, hd, xss, xsd, ea, ei in rels:
        tsrc, tdst = _make_tables(hs, hd, xss, xsd, q)
        eap = _make_eap(ea[_ea_perm], q)
        tables.append((tsrc, tdst, eap))
        npad = _EPT_PAD - _EPT
        es = jnp.pad(ei[0].reshape(_NTILES, _EPT), ((0, 0), (0, npad)))
        ed = jnp.pad(ei[1].reshape(_NTILES, _EPT), ((0, 0), (0, npad)),
                     constant_values=_DUMMY)
        es = es.reshape(_NTILES, _NCHUNK, _CB)
        ed = ed.reshape(_NTILES, _NCHUNK, _CB)
        eis.append(jnp.stack([es, ed], axis=2).reshape(_NTILES * _NCHUNK, 2, _CB))
        wbs.append(q['ese_W2'].T @ q['bw_W'][0])
        wgs.append(q['dg_W2'][0])
        cb = q['ese_b2'] @ q['bw_W'][0] + q['bw_b'][0]
        cg = q['dg_b2'][0]
        cs.append(jnp.zeros((16,), jnp.float32).at[0].set(cb).at[1].set(cg))

    wball = jnp.concatenate(wbs)
    wgall = jnp.concatenate(wgs)
    call = jnp.concatenate(cs)
    zrows = jnp.zeros((_RPS, _H), jnp.float32)

    p1, p2 = _sc_call(tables, eis, wball, wgall, call, zrows)

    P = jnp.stack([p1, p2])
    h_stack = jnp.stack([h_oneD, h_twoD])
    dyn_stack = jnp.stack([dyn_oneD, dyn_twoD])

    def np_(name, f):
        return jnp.stack([f(pn['oneD'][name]), f(pn['twoD'][name])])

    out = _node_update(P, h_stack, dyn_stack, np_)
    return out.reshape(2 * _N, _H)
